# baseline (speedup 1.0000x reference)
import functools

import jax
import jax.numpy as jnp
from jax import lax
from jax.experimental import pallas as pl
from jax.experimental.pallas import tpu as pltpu
from jax.experimental.pallas import tpu_sc as plsc

_N = 10000
_E = 320000
_H = 128

_BN = 2000
_BE = 2560
_CH = 128
_NTILES = 32
_EPT = _E // _NTILES
_NFULL = _EPT // _CH
_REM = _EPT - _NFULL * _CH
_RPT = 624
_RTAIL = _N - 16 * _RPT


def _ln(h):
    mu = jnp.mean(h, axis=-1, keepdims=True)
    d = h - mu
    var = jnp.mean(d * d, axis=-1, keepdims=True)
    return d * lax.rsqrt(var + 1e-5)


def _mm(a, b):
    return jnp.dot(a, b, preferred_element_type=jnp.float32)


def _b16(a):
    return a.astype(jnp.bfloat16).astype(jnp.float32)


def _rsqrt(x):
    y = lax.rsqrt(x)
    return y * (1.5 - 0.5 * x * y * y)


def _wspec(shape):
    return pl.BlockSpec(shape, lambda i: (0,) * len(shape))


def _node_encoder(x2, posp, embp, w1e, w1p, b1, w2, b2, w3, b3):
    def body(x_ref, p_ref, emb_ref, w1e_ref, w1p_ref, b1_ref, w2_ref, b2_ref,
             w3_ref, b3_ref, o_ref, ob_ref):
        iot = lax.broadcasted_iota(jnp.int32, (_BN, 16), 1)
        oh = (iot == x_ref[...]).astype(jnp.float32)
        e = jnp.dot(oh, emb_ref[...], preferred_element_type=jnp.float32,
                    precision=lax.Precision.HIGHEST)
        h = _mm(e, w1e_ref[...]) + _mm(p_ref[...], w1p_ref[...]) + b1_ref[...]
        h = jax.nn.relu(h)
        h = jax.nn.relu(_mm(h, w2_ref[...]) + b2_ref[...])
        h = _mm(h, w3_ref[...]) + b3_ref[...]
        r = _ln(h)
        o_ref[...] = r
        ob_ref[...] = r.astype(jnp.bfloat16)

    return pl.pallas_call(
        body,
        grid=(_N // _BN,),
        in_specs=[
            pl.BlockSpec((_BN, 1), lambda i: (i, 0)),
            pl.BlockSpec((_BN, 24), lambda i: (i, 0)),
            _wspec((16, 16)), _wspec((16, _H)), _wspec((24, _H)),
            _wspec((1, _H)), _wspec((_H, _H)), _wspec((1, _H)),
            _wspec((_H, _H)), _wspec((1, _H)),
        ],
        out_specs=[pl.BlockSpec((_BN, _H), lambda i: (i, 0)),
                   pl.BlockSpec((_BN, _H), lambda i: (i, 0))],
        out_shape=[jax.ShapeDtypeStruct((_N, _H), jnp.float32),
                   jax.ShapeDtypeStruct((_N, _H), jnp.bfloat16)],
    )(x2, posp, embp, w1e, w1p, b1, w2, b2, w3, b3)


def _edge_encoder(eap, w1, b1, w2, b2, w3, b3):
    def body(a_ref, w1_ref, b1_ref, w2_ref, b2_ref, w3_ref, b3_ref, o_ref):
        h = jax.nn.relu(_mm(a_ref[...], w1_ref[...]) + b1_ref[...])
        h = jax.nn.relu(_mm(h, w2_ref[...]) + b2_ref[...])
        h = _mm(h, w3_ref[...]) + b3_ref[...]
        o_ref[...] = _ln(h)

    return pl.pallas_call(
        body,
        grid=(_E // _BE,),
        in_specs=[
            pl.BlockSpec((_BE, 8), lambda i: (i, 0)),
            _wspec((8, _H)), _wspec((1, _H)), _wspec((_H, _H)),
            _wspec((1, _H)), _wspec((_H, _H)), _wspec((1, _H)),
        ],
        out_specs=pl.BlockSpec((_BE, _H), lambda i: (i, 0)),
        out_shape=jax.ShapeDtypeStruct((_E, _H), jnp.float32),
    )(eap, w1, b1, w2, b2, w3, b3)


def _sc_gather2(nf_pk, src, dst):
    mesh = plsc.VectorSubcoreMesh(core_axis_name="c", subcore_axis_name="s")
    _HP = _H // 2

    @functools.partial(
        pl.kernel,
        out_type=(jax.ShapeDtypeStruct((_E, _HP), jnp.int32),
                  jax.ShapeDtypeStruct((_E, _HP), jnp.int32)),
        mesh=mesh,
        scratch_types=[
            pltpu.VMEM((_CH,), jnp.int32), pltpu.VMEM((_CH,), jnp.int32),
            pltpu.VMEM((_CH, _HP), jnp.int32), pltpu.VMEM((_CH, _HP), jnp.int32),
            pltpu.VMEM((_REM,), jnp.int32), pltpu.VMEM((_REM,), jnp.int32),
            pltpu.VMEM((_REM, _HP), jnp.int32), pltpu.VMEM((_REM, _HP), jnp.int32),
        ],
        compiler_params=pltpu.CompilerParams(use_tc_tiling_on_sc=False),
    )
    def k(nf_hbm, src_hbm, dst_hbm, xi_hbm, xj_hbm,
          sidx, didx, srows, drows, sidx2, didx2, srows2, drows2):
        wid = lax.axis_index("s") * 2 + lax.axis_index("c")
        base0 = wid * _EPT

        def body(i, carry):
            base = base0 + i * _CH
            pltpu.sync_copy(dst_hbm.at[pl.ds(base, _CH)], didx)
            pltpu.sync_copy(src_hbm.at[pl.ds(base, _CH)], sidx)
            pltpu.sync_copy(nf_hbm.at[didx], drows)
            pltpu.sync_copy(nf_hbm.at[sidx], srows)
            pltpu.sync_copy(drows, xi_hbm.at[pl.ds(base, _CH)])
            pltpu.sync_copy(srows, xj_hbm.at[pl.ds(base, _CH)])
            return carry

        lax.fori_loop(0, _NFULL, body, 0)
        base = base0 + _NFULL * _CH
        pltpu.sync_copy(dst_hbm.at[pl.ds(base, _REM)], didx2)
        pltpu.sync_copy(src_hbm.at[pl.ds(base, _REM)], sidx2)
        pltpu.sync_copy(nf_hbm.at[didx2], drows2)
        pltpu.sync_copy(nf_hbm.at[sidx2], srows2)
        pltpu.sync_copy(drows2, xi_hbm.at[pl.ds(base, _REM)])
        pltpu.sync_copy(srows2, xj_hbm.at[pl.ds(base, _REM)])

    return k(nf_pk, src, dst)


def _sc_segsum(m, dst, zeros_nh):
    mesh = plsc.VectorSubcoreMesh(core_axis_name="c", subcore_axis_name="s")

    @functools.partial(
        pl.kernel,
        out_type=jax.ShapeDtypeStruct((2, _N, _H), jnp.float32),
        mesh=mesh,
        scratch_types=[
            pltpu.VMEM((_CH,), jnp.int32), pltpu.VMEM((_CH, _H), jnp.float32),
            pltpu.VMEM((_REM,), jnp.int32), pltpu.VMEM((_REM, _H), jnp.float32),
            pltpu.VMEM_SHARED((_N, _H), jnp.float32),
        ],
    )
    def k(m_hbm, dst_hbm, z_hbm, out_hbm, idx, buf, idx2, buf2, acc):
        cid = lax.axis_index("c")
        sid = lax.axis_index("s")
        wid = sid * 2 + cid
        r0 = sid * _RPT
        pltpu.sync_copy(z_hbm.at[pl.ds(r0, _RPT)], acc.at[pl.ds(r0, _RPT)])

        @pl.when(sid == 15)
        def _():
            pltpu.sync_copy(z_hbm.at[pl.ds(16 * _RPT, _RTAIL)],
                            acc.at[pl.ds(16 * _RPT, _RTAIL)])

        plsc.subcore_barrier()
        base0 = wid * _EPT

        def body(i, carry):
            base = base0 + i * _CH
            pltpu.sync_copy(dst_hbm.at[pl.ds(base, _CH)], idx)
            pltpu.sync_copy(m_hbm.at[pl.ds(base, _CH)], buf)
            pltpu.sync_copy(buf, acc.at[idx], add=True)
            return carry

        lax.fori_loop(0, _NFULL, body, 0)
        base = base0 + _NFULL * _CH
        pltpu.sync_copy(dst_hbm.at[pl.ds(base, _REM)], idx2)
        pltpu.sync_copy(m_hbm.at[pl.ds(base, _REM)], buf2)
        pltpu.sync_copy(buf2, acc.at[idx2], add=True)
        plsc.subcore_barrier()
        pltpu.sync_copy(acc.at[pl.ds(r0, _RPT)], out_hbm.at[cid, pl.ds(r0, _RPT)])

        @pl.when(sid == 15)
        def _():
            pltpu.sync_copy(acc.at[pl.ds(16 * _RPT, _RTAIL)],
                            out_hbm.at[cid, pl.ds(16 * _RPT, _RTAIL)])

    return k(m, dst, zeros_nh)


def _edge_layer(xi, xj, ef, nd2, w1a, w1b, w1c, b1, w2, b2, w3, b3,
                dw1, db1, dw2, db2):
    def body(xi_ref, xj_ref, ef_ref, nd_ref, w1a_ref, w1b_ref, w1c_ref, b1_ref,
             w2_ref, b2_ref, w3_ref, b3_ref, dw1_ref, db1_ref, dw2_ref, db2_ref,
             m_ref, eo_ref):
        ef_v = ef_ref[...]
        h = (_mm(xi_ref[...], w1a_ref[...]) + _mm(xj_ref[...], w1b_ref[...])
             + _mm(ef_v, w1c_ref[...]) + b1_ref[...])
        h = jax.nn.relu(h)
        h = jax.nn.relu(_mm(h, w2_ref[...]) + b2_ref[...])
        h = _mm(h, w3_ref[...]) + b3_ref[...]
        t = jax.nn.relu(nd_ref[...] * dw1_ref[...] + db1_ref[...])
        w = _mm(t, dw2_ref[...]) + db2_ref[...]
        m = _ln(h) * w
        m_ref[...] = m
        eo_ref[...] = ef_v + m

    return pl.pallas_call(
        body,
        grid=(_E // _BE,),
        in_specs=[
            pl.BlockSpec((_BE, _H), lambda i: (i, 0)),
            pl.BlockSpec((_BE, _H), lambda i: (i, 0)),
            pl.BlockSpec((_BE, _H), lambda i: (i, 0)),
            pl.BlockSpec((_BE, 1), lambda i: (i, 0)),
            _wspec((_H, _H)), _wspec((_H, _H)), _wspec((_H, _H)), _wspec((1, _H)),
            _wspec((_H, _H)), _wspec((1, _H)), _wspec((_H, _H)), _wspec((1, _H)),
            _wspec((1, _H)), _wspec((1, _H)), _wspec((_H, _H)), _wspec((1, _H)),
        ],
        out_specs=[
            pl.BlockSpec((_BE, _H), lambda i: (i, 0)),
            pl.BlockSpec((_BE, _H), lambda i: (i, 0)),
        ],
        out_shape=[
            jax.ShapeDtypeStruct((_E, _H), jnp.float32),
            jax.ShapeDtypeStruct((_E, _H), jnp.float32),
        ],
    )(xi, xj, ef, nd2, w1a, w1b, w1c, b1, w2, b2, w3, b3, dw1, db1, dw2, db2)


def _node_layer(nf, parts, wna, wnb, b1, w2, b2, w3, b3):
    def body(nf_ref, pr_ref, wna_ref, wnb_ref, b1_ref, w2_ref, b2_ref,
             w3_ref, b3_ref, o_ref, ob_ref):
        nf_v = nf_ref[...]
        aggr = pr_ref[0] + pr_ref[1]
        h = _mm(nf_v, wna_ref[...]) + _mm(aggr, wnb_ref[...]) + b1_ref[...]
        h = jax.nn.relu(h)
        h = jax.nn.relu(_mm(h, w2_ref[...]) + b2_ref[...])
        h = _mm(h, w3_ref[...]) + b3_ref[...]
        r = nf_v + _ln(h)
        o_ref[...] = r
        ob_ref[...] = r.astype(jnp.bfloat16)

    return pl.pallas_call(
        body,
        grid=(_N // _BN,),
        in_specs=[
            pl.BlockSpec((_BN, _H), lambda i: (i, 0)),
            pl.BlockSpec((2, _BN, _H), lambda i: (0, i, 0)),
            _wspec((_H, _H)), _wspec((_H, _H)), _wspec((1, _H)),
            _wspec((_H, _H)), _wspec((1, _H)), _wspec((_H, _H)), _wspec((1, _H)),
        ],
        out_specs=[pl.BlockSpec((_BN, _H), lambda i: (i, 0)),
                   pl.BlockSpec((_BN, _H), lambda i: (i, 0))],
        out_shape=[jax.ShapeDtypeStruct((_N, _H), jnp.float32),
                   jax.ShapeDtypeStruct((_N, _H), jnp.bfloat16)],
    )(nf, parts, wna, wnb, b1, w2, b2, w3, b3)


def _decoder(nf, w1, b1, w2, b2, w3p, b3p):
    def body(nf_ref, w1_ref, b1_ref, w2_ref, b2_ref, w3_ref, b3_ref, o_ref):
        h = jax.nn.relu(_mm(nf_ref[...], w1_ref[...]) + b1_ref[...])
        h = jax.nn.relu(_mm(h, w2_ref[...]) + b2_ref[...])
        o_ref[...] = _mm(h, w3_ref[...]) + b3_ref[...]

    return pl.pallas_call(
        body,
        grid=(_N // _BN,),
        in_specs=[
            pl.BlockSpec((_BN, _H), lambda i: (i, 0)),
            _wspec((_H, _H)), _wspec((1, _H)), _wspec((_H, _H)),
            _wspec((1, _H)), _wspec((_H, 8)), _wspec((1, 8)),
        ],
        out_specs=pl.BlockSpec((_BN, 8), lambda i: (i, 0)),
        out_shape=jax.ShapeDtypeStruct((_N, 8), jnp.float32),
    )(nf, w1, b1, w2, b2, w3p, b3p)


def _r(b):
    return b.reshape(1, -1)


def _pack(nf_bf):
    return lax.bitcast_convert_type(nf_bf.reshape(_N, _H // 2, 2), jnp.int32)


def _unpack(x_pk):
    return lax.bitcast_convert_type(x_pk, jnp.bfloat16).reshape(_E, _H)


def kernel(x, pos, edge_index, edge_attr, node_dist, params):
    src = edge_index[0].astype(jnp.int32)
    dst = edge_index[1].astype(jnp.int32)
    x2 = x.astype(jnp.int32).reshape(_N, 1)
    posp = jnp.pad(pos, ((0, 0), (0, 3)))
    eap = jnp.pad(edge_attr, ((0, 0), (0, 4)))
    nd2 = node_dist.reshape(_E, 1)
    zeros_nh = jnp.zeros((_N, _H), jnp.float32)

    embp = jnp.pad(params['embed'], ((0, 16 - params['embed'].shape[0]), (0, 0)))
    (w1, b1), (w2, b2), (w3, b3) = params['node_in']
    nf, nf_bf = _node_encoder(x2, posp, embp, w1[:16],
                              jnp.pad(w1[16:], ((0, 3), (0, 0))),
                              _r(b1), w2, _r(b2), w3, _r(b3))

    (ew1, eb1), (ew2, eb2), (ew3, eb3) = params['edge_in']
    ef = _edge_encoder(eap, jnp.pad(ew1, ((0, 4), (0, 0))), _r(eb1),
                       ew2, _r(eb2), ew3, _r(eb3))

    for p in params['layers']:
        (a1, ab1), (a2, ab2), (a3, ab3) = p['edge_mlp']
        (d1, db1), (d2, db2) = p['dist']
        (n1, nb1), (n2, nb2), (n3, nb3) = p['node_mlp']
        xi_pk, xj_pk = _sc_gather2(_pack(nf_bf), src, dst)
        xi, xj = _unpack(xi_pk), _unpack(xj_pk)
        m, ef = _edge_layer(xi, xj, ef, nd2,
                            a1[:_H], a1[_H:2 * _H], a1[2 * _H:], _r(ab1),
                            a2, _r(ab2), a3, _r(ab3),
                            d1, _r(db1), d2, _r(db2))
        parts = _sc_segsum(m, dst, zeros_nh)
        nf, nf_bf = _node_layer(nf, parts, n1[:_H], n1[_H:], _r(nb1),
                                n2, _r(nb2), n3, _r(nb3))

    (o1, ob1), (o2, ob2), (o3, ob3) = params['node_out']
    out = _decoder(nf, o1, _r(ob1), o2, _r(ob2),
                   jnp.pad(o3, ((0, 0), (0, 5))), jnp.pad(_r(ob3), ((0, 0), (0, 5))))
    return out[:, :3]


# speedup vs baseline: 2.8912x; 2.8912x over previous
import functools

import jax
import jax.numpy as jnp
from jax import lax
from jax.experimental import pallas as pl
from jax.experimental.pallas import tpu as pltpu
from jax.experimental.pallas import tpu_sc as plsc

_N = 10000
_E = 320000
_H = 128

_BN = 2000
_BE = 2560
_CH = 128
_NTILES = 32
_EPT = _E // _NTILES
_NFULL = _EPT // _CH
_REM = _EPT - _NFULL * _CH
_RPT = 624
_RTAIL = _N - 16 * _RPT


def _ln(h):
    mu = jnp.mean(h, axis=-1, keepdims=True)
    d = h - mu
    var = jnp.mean(d * d, axis=-1, keepdims=True)
    return d * lax.rsqrt(var + 1e-5)


def _mm(a, b):
    return jnp.dot(a, b, preferred_element_type=jnp.float32)


def _b16(a):
    return a.astype(jnp.bfloat16).astype(jnp.float32)


def _rsqrt(x):
    y = lax.rsqrt(x)
    return y * (1.5 - 0.5 * x * y * y)


def _wspec(shape):
    return pl.BlockSpec(shape, lambda i: (0,) * len(shape))


def _node_encoder(x2, posp, embp, w1e, w1p, b1, w2, b2, w3, b3):
    def body(x_ref, p_ref, emb_ref, w1e_ref, w1p_ref, b1_ref, w2_ref, b2_ref,
             w3_ref, b3_ref, o_ref, ob_ref):
        iot = lax.broadcasted_iota(jnp.int32, (_BN, 16), 1)
        oh = (iot == x_ref[...]).astype(jnp.float32)
        e = jnp.dot(oh, emb_ref[...], preferred_element_type=jnp.float32,
                    precision=lax.Precision.HIGHEST)
        h = _mm(e, w1e_ref[...]) + _mm(p_ref[...], w1p_ref[...]) + b1_ref[...]
        h = jax.nn.relu(h)
        h = jax.nn.relu(_mm(h, w2_ref[...]) + b2_ref[...])
        h = _mm(h, w3_ref[...]) + b3_ref[...]
        r = _ln(h)
        o_ref[...] = r
        ob_ref[...] = r.astype(jnp.bfloat16)

    return pl.pallas_call(
        body,
        grid=(_N // _BN,),
        in_specs=[
            pl.BlockSpec((_BN, 1), lambda i: (i, 0)),
            pl.BlockSpec((_BN, 24), lambda i: (i, 0)),
            _wspec((16, 16)), _wspec((16, _H)), _wspec((24, _H)),
            _wspec((1, _H)), _wspec((_H, _H)), _wspec((1, _H)),
            _wspec((_H, _H)), _wspec((1, _H)),
        ],
        out_specs=[pl.BlockSpec((_BN, _H), lambda i: (i, 0)),
                   pl.BlockSpec((_BN, _H), lambda i: (i, 0))],
        out_shape=[jax.ShapeDtypeStruct((_N, _H), jnp.float32),
                   jax.ShapeDtypeStruct((_N, _H), jnp.bfloat16)],
    )(x2, posp, embp, w1e, w1p, b1, w2, b2, w3, b3)


def _edge_encoder(eap, w1, b1, w2, b2, w3, b3):
    def body(a_ref, w1_ref, b1_ref, w2_ref, b2_ref, w3_ref, b3_ref, o_ref):
        h = jax.nn.relu(_mm(a_ref[...], w1_ref[...]) + b1_ref[...])
        h = jax.nn.relu(_mm(h, w2_ref[...]) + b2_ref[...])
        h = _mm(h, w3_ref[...]) + b3_ref[...]
        o_ref[...] = _ln(h)

    return pl.pallas_call(
        body,
        grid=(_E // _BE,),
        in_specs=[
            pl.BlockSpec((_BE, 8), lambda i: (i, 0)),
            _wspec((8, _H)), _wspec((1, _H)), _wspec((_H, _H)),
            _wspec((1, _H)), _wspec((_H, _H)), _wspec((1, _H)),
        ],
        out_specs=pl.BlockSpec((_BE, _H), lambda i: (i, 0)),
        out_shape=jax.ShapeDtypeStruct((_E, _H), jnp.float32),
    )(eap, w1, b1, w2, b2, w3, b3)


def _sc_gather2(nf, src, dst):
    mesh = plsc.VectorSubcoreMesh(core_axis_name="c", subcore_axis_name="s")

    @functools.partial(
        pl.kernel,
        out_type=(jax.ShapeDtypeStruct((_E, _H), jnp.float32),
                  jax.ShapeDtypeStruct((_E, _H), jnp.float32)),
        mesh=mesh,
        scratch_types=[
            pltpu.VMEM((_EPT,), jnp.int32), pltpu.VMEM((_EPT,), jnp.int32),
            pltpu.VMEM((3, _CH, _H), jnp.float32),
            pltpu.VMEM((3, _CH, _H), jnp.float32),
            pltpu.SemaphoreType.DMA, pltpu.SemaphoreType.DMA,
            pltpu.SemaphoreType.DMA, pltpu.SemaphoreType.DMA,
            pltpu.SemaphoreType.DMA, pltpu.SemaphoreType.DMA,
        ],
    )
    def k(nf_hbm, src_hbm, dst_hbm, xi_hbm, xj_hbm,
          sidx, didx, srows, drows, gs0, gs1, gs2, os0, os1, os2):
        wid = lax.axis_index("s") * 2 + lax.axis_index("c")
        base0 = wid * _EPT
        pltpu.sync_copy(src_hbm.at[pl.ds(base0, _EPT)], sidx)
        pltpu.sync_copy(dst_hbm.at[pl.ds(base0, _EPT)], didx)
        gsem = [gs0, gs1, gs2]
        osem = [os0, os1, os2]

        def gather(i, b, n):
            o = i * _CH
            pltpu.async_copy(nf_hbm.at[didx.at[pl.ds(o, n)]],
                             drows.at[b, pl.ds(0, n)], gsem[b])
            pltpu.async_copy(nf_hbm.at[sidx.at[pl.ds(o, n)]],
                             srows.at[b, pl.ds(0, n)], gsem[b])

        def wait_gather(b, n):
            pltpu.make_async_copy(nf_hbm.at[didx.at[pl.ds(0, n)]],
                                  drows.at[b, pl.ds(0, n)], gsem[b]).wait()
            pltpu.make_async_copy(nf_hbm.at[sidx.at[pl.ds(0, n)]],
                                  srows.at[b, pl.ds(0, n)], gsem[b]).wait()

        def writeback(i, b, n):
            base = base0 + i * _CH
            pltpu.async_copy(drows.at[b, pl.ds(0, n)],
                             xi_hbm.at[pl.ds(base, n)], osem[b])
            pltpu.async_copy(srows.at[b, pl.ds(0, n)],
                             xj_hbm.at[pl.ds(base, n)], osem[b])

        def wait_writeback(b, n):
            pltpu.make_async_copy(drows.at[b, pl.ds(0, n)],
                                  xi_hbm.at[pl.ds(base0, n)], osem[b]).wait()
            pltpu.make_async_copy(srows.at[b, pl.ds(0, n)],
                                  xj_hbm.at[pl.ds(base0, n)], osem[b]).wait()

        def tri(j, carry):
            for b in range(3):
                @pl.when(j > 0)
                def _(b=b):
                    wait_writeback(b, _CH)

                gather(3 * j + b, b, _CH)
            for b in range(3):
                wait_gather(b, _CH)
                writeback(3 * j + b, b, _CH)
            return carry

        lax.fori_loop(0, _NFULL // 3, tri, 0)
        for b in range(3):
            wait_writeback(b, _CH)
        gather(_NFULL, 0, _REM)
        wait_gather(0, _REM)
        writeback(_NFULL, 0, _REM)
        wait_writeback(0, _REM)

    return k(nf, src, dst)


def _sc_segsum(m, dst, zeros_nh):
    mesh = plsc.VectorSubcoreMesh(core_axis_name="c", subcore_axis_name="s")

    @functools.partial(
        pl.kernel,
        out_type=jax.ShapeDtypeStruct((2, _N, _H), jnp.float32),
        mesh=mesh,
        scratch_types=[
            pltpu.VMEM((_CH,), jnp.int32), pltpu.VMEM((_CH, _H), jnp.float32),
            pltpu.VMEM((_REM,), jnp.int32), pltpu.VMEM((_REM, _H), jnp.float32),
            pltpu.VMEM_SHARED((_N, _H), jnp.float32),
        ],
    )
    def k(m_hbm, dst_hbm, z_hbm, out_hbm, idx, buf, idx2, buf2, acc):
        cid = lax.axis_index("c")
        sid = lax.axis_index("s")
        wid = sid * 2 + cid
        r0 = sid * _RPT
        pltpu.sync_copy(z_hbm.at[pl.ds(r0, _RPT)], acc.at[pl.ds(r0, _RPT)])

        @pl.when(sid == 15)
        def _():
            pltpu.sync_copy(z_hbm.at[pl.ds(16 * _RPT, _RTAIL)],
                            acc.at[pl.ds(16 * _RPT, _RTAIL)])

        plsc.subcore_barrier()
        base0 = wid * _EPT

        def body(i, carry):
            base = base0 + i * _CH
            pltpu.sync_copy(dst_hbm.at[pl.ds(base, _CH)], idx)
            pltpu.sync_copy(m_hbm.at[pl.ds(base, _CH)], buf)
            pltpu.sync_copy(buf, acc.at[idx], add=True)
            return carry

        lax.fori_loop(0, _NFULL, body, 0)
        base = base0 + _NFULL * _CH
        pltpu.sync_copy(dst_hbm.at[pl.ds(base, _REM)], idx2)
        pltpu.sync_copy(m_hbm.at[pl.ds(base, _REM)], buf2)
        pltpu.sync_copy(buf2, acc.at[idx2], add=True)
        plsc.subcore_barrier()
        pltpu.sync_copy(acc.at[pl.ds(r0, _RPT)], out_hbm.at[cid, pl.ds(r0, _RPT)])

        @pl.when(sid == 15)
        def _():
            pltpu.sync_copy(acc.at[pl.ds(16 * _RPT, _RTAIL)],
                            out_hbm.at[cid, pl.ds(16 * _RPT, _RTAIL)])

    return k(m, dst, zeros_nh)


def _edge_layer(xi, xj, ef, nd2, w1a, w1b, w1c, b1, w2, b2, w3, b3,
                dw1, db1, dw2, db2):
    def body(xi_ref, xj_ref, ef_ref, nd_ref, w1a_ref, w1b_ref, w1c_ref, b1_ref,
             w2_ref, b2_ref, w3_ref, b3_ref, dw1_ref, db1_ref, dw2_ref, db2_ref,
             m_ref, eo_ref):
        ef_v = ef_ref[...]
        h = (_mm(xi_ref[...], w1a_ref[...]) + _mm(xj_ref[...], w1b_ref[...])
             + _mm(ef_v, w1c_ref[...]) + b1_ref[...])
        h = jax.nn.relu(h)
        h = jax.nn.relu(_mm(h, w2_ref[...]) + b2_ref[...])
        h = _mm(h, w3_ref[...]) + b3_ref[...]
        t = jax.nn.relu(nd_ref[...] * dw1_ref[...] + db1_ref[...])
        w = _mm(t, dw2_ref[...]) + db2_ref[...]
        m = _ln(h) * w
        m_ref[...] = m
        eo_ref[...] = ef_v + m

    return pl.pallas_call(
        body,
        grid=(_E // _BE,),
        in_specs=[
            pl.BlockSpec((_BE, _H), lambda i: (i, 0)),
            pl.BlockSpec((_BE, _H), lambda i: (i, 0)),
            pl.BlockSpec((_BE, _H), lambda i: (i, 0)),
            pl.BlockSpec((_BE, 1), lambda i: (i, 0)),
            _wspec((_H, _H)), _wspec((_H, _H)), _wspec((_H, _H)), _wspec((1, _H)),
            _wspec((_H, _H)), _wspec((1, _H)), _wspec((_H, _H)), _wspec((1, _H)),
            _wspec((1, _H)), _wspec((1, _H)), _wspec((_H, _H)), _wspec((1, _H)),
        ],
        out_specs=[
            pl.BlockSpec((_BE, _H), lambda i: (i, 0)),
            pl.BlockSpec((_BE, _H), lambda i: (i, 0)),
        ],
        out_shape=[
            jax.ShapeDtypeStruct((_E, _H), jnp.float32),
            jax.ShapeDtypeStruct((_E, _H), jnp.float32),
        ],
    )(xi, xj, ef, nd2, w1a, w1b, w1c, b1, w2, b2, w3, b3, dw1, db1, dw2, db2)


def _node_layer(nf, parts, wna, wnb, b1, w2, b2, w3, b3):
    def body(nf_ref, pr_ref, wna_ref, wnb_ref, b1_ref, w2_ref, b2_ref,
             w3_ref, b3_ref, o_ref, ob_ref):
        nf_v = nf_ref[...]
        aggr = pr_ref[0] + pr_ref[1]
        h = _mm(nf_v, wna_ref[...]) + _mm(aggr, wnb_ref[...]) + b1_ref[...]
        h = jax.nn.relu(h)
        h = jax.nn.relu(_mm(h, w2_ref[...]) + b2_ref[...])
        h = _mm(h, w3_ref[...]) + b3_ref[...]
        r = nf_v + _ln(h)
        o_ref[...] = r
        ob_ref[...] = r.astype(jnp.bfloat16)

    return pl.pallas_call(
        body,
        grid=(_N // _BN,),
        in_specs=[
            pl.BlockSpec((_BN, _H), lambda i: (i, 0)),
            pl.BlockSpec((2, _BN, _H), lambda i: (0, i, 0)),
            _wspec((_H, _H)), _wspec((_H, _H)), _wspec((1, _H)),
            _wspec((_H, _H)), _wspec((1, _H)), _wspec((_H, _H)), _wspec((1, _H)),
        ],
        out_specs=[pl.BlockSpec((_BN, _H), lambda i: (i, 0)),
                   pl.BlockSpec((_BN, _H), lambda i: (i, 0))],
        out_shape=[jax.ShapeDtypeStruct((_N, _H), jnp.float32),
                   jax.ShapeDtypeStruct((_N, _H), jnp.bfloat16)],
    )(nf, parts, wna, wnb, b1, w2, b2, w3, b3)


def _decoder(nf, w1, b1, w2, b2, w3p, b3p):
    def body(nf_ref, w1_ref, b1_ref, w2_ref, b2_ref, w3_ref, b3_ref, o_ref):
        h = jax.nn.relu(_mm(nf_ref[...], w1_ref[...]) + b1_ref[...])
        h = jax.nn.relu(_mm(h, w2_ref[...]) + b2_ref[...])
        o_ref[...] = _mm(h, w3_ref[...]) + b3_ref[...]

    return pl.pallas_call(
        body,
        grid=(_N // _BN,),
        in_specs=[
            pl.BlockSpec((_BN, _H), lambda i: (i, 0)),
            _wspec((_H, _H)), _wspec((1, _H)), _wspec((_H, _H)),
            _wspec((1, _H)), _wspec((_H, 8)), _wspec((1, 8)),
        ],
        out_specs=pl.BlockSpec((_BN, 8), lambda i: (i, 0)),
        out_shape=jax.ShapeDtypeStruct((_N, 8), jnp.float32),
    )(nf, w1, b1, w2, b2, w3p, b3p)


def _r(b):
    return b.reshape(1, -1)


def _pack(nf_bf):
    return lax.bitcast_convert_type(nf_bf.reshape(_N, _H // 2, 2), jnp.int32)


def _unpack(x_pk):
    return lax.bitcast_convert_type(x_pk, jnp.bfloat16).reshape(_E, _H)


def kernel(x, pos, edge_index, edge_attr, node_dist, params):
    src = edge_index[0].astype(jnp.int32)
    dst = edge_index[1].astype(jnp.int32)
    x2 = x.astype(jnp.int32).reshape(_N, 1)
    posp = jnp.pad(pos, ((0, 0), (0, 3)))
    eap = jnp.pad(edge_attr, ((0, 0), (0, 4)))
    nd2 = node_dist.reshape(_E, 1)
    zeros_nh = jnp.zeros((_N, _H), jnp.float32)

    embp = jnp.pad(params['embed'], ((0, 16 - params['embed'].shape[0]), (0, 0)))
    (w1, b1), (w2, b2), (w3, b3) = params['node_in']
    nf, nf_bf = _node_encoder(x2, posp, embp, w1[:16],
                              jnp.pad(w1[16:], ((0, 3), (0, 0))),
                              _r(b1), w2, _r(b2), w3, _r(b3))

    (ew1, eb1), (ew2, eb2), (ew3, eb3) = params['edge_in']
    ef = _edge_encoder(eap, jnp.pad(ew1, ((0, 4), (0, 0))), _r(eb1),
                       ew2, _r(eb2), ew3, _r(eb3))

    for p in params['layers']:
        (a1, ab1), (a2, ab2), (a3, ab3) = p['edge_mlp']
        (d1, db1), (d2, db2) = p['dist']
        (n1, nb1), (n2, nb2), (n3, nb3) = p['node_mlp']
        xi, xj = _sc_gather2(nf, src, dst)
        m, ef = _edge_layer(xi, xj, ef, nd2,
                            a1[:_H], a1[_H:2 * _H], a1[2 * _H:], _r(ab1),
                            a2, _r(ab2), a3, _r(ab3),
                            d1, _r(db1), d2, _r(db2))
        parts = _sc_segsum(m, dst, zeros_nh)
        nf, nf_bf = _node_layer(nf, parts, n1[:_H], n1[_H:], _r(nb1),
                                n2, _r(nb2), n3, _r(nb3))

    (o1, ob1), (o2, ob2), (o3, ob3) = params['node_out']
    out = _decoder(nf, o1, _r(ob1), o2, _r(ob2),
                   jnp.pad(o3, ((0, 0), (0, 5))), jnp.pad(_r(ob3), ((0, 0), (0, 5))))
    return out[:, :3]
